# baseline (speedup 1.0000x reference)
import functools
import math

import jax
import jax.numpy as jnp
from jax import lax
from jax.experimental import pallas as pl
from jax.experimental.pallas import tpu as pltpu
from jax.experimental.pallas import tpu_sc as plsc

VOCAB = 1000000
D = 64
BATCH = 1024
SEQ = 200

NC = 2
NS = 16
NW = NC * NS
ROWS_PER_W = BATCH // NW
HALF = SEQ // 2

_SCALE = math.sqrt(D)


def _pe_body(out_ref):
    pos = lax.broadcasted_iota(jnp.int32, (SEQ, D), 0).astype(jnp.float32)
    i = lax.broadcasted_iota(jnp.int32, (SEQ, D), 1)
    two_i = (2 * (i // 2)).astype(jnp.float32)
    inv_rate = jnp.exp(-(math.log(10000.0) / D) * two_i)
    angles = pos * inv_rate
    even = (i % 2) == 0
    out_ref[...] = jnp.where(even, jnp.sin(angles), jnp.cos(angles))


def _positional_encoding():
    return pl.pallas_call(
        _pe_body,
        out_shape=jax.ShapeDtypeStruct((SEQ, D), jnp.float32),
    )()


def _sc_body(x_hbm, table_hbm, pe_hbm, out_hbm, idx_v, rows_v, pe_v, sem):
    cid = lax.axis_index("c")
    sid = lax.axis_index("s")
    wid = sid * NC + cid
    b0 = wid * ROWS_PER_W

    pltpu.sync_copy(pe_hbm, pe_v)

    def row_body(b, carry):
        babs = b0 + b
        pltpu.sync_copy(x_hbm.at[babs], idx_v)
        cp0 = pltpu.async_copy(
            table_hbm.at[idx_v.at[0]], rows_v.at[pl.ds(0, HALF)], sem)
        cp1 = pltpu.async_copy(
            table_hbm.at[idx_v.at[1]], rows_v.at[pl.ds(HALF, HALF)], sem)
        cp0.wait()
        cp1.wait()

        def fma_body(r, c2):
            for c in range(D // 16):
                sl = (r, pl.ds(c * 16, 16))
                rows_v[sl] = rows_v[sl] * _SCALE + pe_v[sl]
            return c2

        lax.fori_loop(0, SEQ, fma_body, 0)
        pltpu.sync_copy(rows_v, out_hbm.at[babs])
        return carry

    lax.fori_loop(0, ROWS_PER_W, row_body, 0)


@functools.partial(jax.jit, static_argnames=())
def _embed(x3, table, pe):
    mesh = plsc.VectorSubcoreMesh(core_axis_name="c", subcore_axis_name="s")
    fn = functools.partial(
        pl.kernel,
        mesh=mesh,
        out_type=jax.ShapeDtypeStruct((BATCH, SEQ, D), jnp.float32),
        scratch_types=[
            pltpu.VMEM((2, HALF), jnp.int32),
            pltpu.VMEM((SEQ, D), jnp.float32),
            pltpu.VMEM((SEQ, D), jnp.float32),
            pltpu.SemaphoreType.DMA,
        ],
        compiler_params=pltpu.CompilerParams(use_tc_tiling_on_sc=False),
    )(_sc_body)
    return fn(x3, table, pe)


def kernel(x, table):
    pe = _positional_encoding()
    x3 = x.reshape(BATCH, 2, HALF)
    return _embed(x3, table, pe)


# speedup vs baseline: 1.0767x; 1.0767x over previous
import functools
import math

import jax
import jax.numpy as jnp
from jax import lax
from jax.experimental import pallas as pl
from jax.experimental.pallas import tpu as pltpu
from jax.experimental.pallas import tpu_sc as plsc

VOCAB = 1000000
D = 64
BATCH = 1024
SEQ = 200

NC = 2
NS = 16
NW = NC * NS
ROWS_PER_W = BATCH // NW
HALF = SEQ // 2

CH = 2
NCHUNK = ROWS_PER_W // CH

_SCALE = math.sqrt(D)


def _pe_body(out_ref):
    pos = lax.broadcasted_iota(jnp.int32, (SEQ, D), 0).astype(jnp.float32)
    i = lax.broadcasted_iota(jnp.int32, (SEQ, D), 1)
    two_i = (2 * (i // 2)).astype(jnp.float32)
    inv_rate = jnp.exp(-(math.log(10000.0) / D) * two_i)
    angles = pos * inv_rate
    even = (i % 2) == 0
    out_ref[...] = jnp.where(even, jnp.sin(angles), jnp.cos(angles))


def _positional_encoding():
    return pl.pallas_call(
        _pe_body,
        out_shape=jax.ShapeDtypeStruct((SEQ, D), jnp.float32),
    )()


def _sc_body(x_hbm, table_hbm, pe_hbm, out_hbm,
             idx_v, buf0, buf1, buf2, pe_v, gsem, osem):
    cid = lax.axis_index("c")
    sid = lax.axis_index("s")
    wid = sid * NC + cid
    b0 = wid * ROWS_PER_W

    bufs = (buf0, buf1, buf2)

    pltpu.sync_copy(x_hbm.at[pl.ds(b0, ROWS_PER_W)], idx_v)
    pltpu.sync_copy(pe_hbm, pe_v)

    def start_gathers(t):
        buf = bufs[t % 3]
        cps = []
        for cc in range(CH):
            for j in range(2):
                cps.append(pltpu.async_copy(
                    table_hbm.at[idx_v.at[t * CH + cc, j]],
                    buf.at[pl.ds((cc * 2 + j) * HALF, HALF)],
                    gsem))
        return cps

    def start_out(t):
        return pltpu.async_copy(
            bufs[t % 3], out_hbm.at[pl.ds((b0 + t * CH) * SEQ, CH * SEQ)],
            osem)

    def compute(t):
        buf = bufs[t % 3]

        @plsc.parallel_loop(0, SEQ, unroll=4)
        def _(r):
            for cc in range(CH):
                for c in range(D // 16):
                    sl = (cc * SEQ + r, pl.ds(c * 16, 16))
                    buf[sl] = buf[sl] * _SCALE + pe_v[r, pl.ds(c * 16, 16)]

    gathers = {0: start_gathers(0), 1: start_gathers(1)}
    outs = {}
    for t in range(NCHUNK):
        for cp in gathers.pop(t):
            cp.wait()
        if t + 2 < NCHUNK:
            if t >= 1:
                outs.pop(t - 1).wait()
            gathers[t + 2] = start_gathers(t + 2)
        compute(t)
        outs[t] = start_out(t)
    for t in sorted(outs):
        outs.pop(t).wait()


@jax.jit
def _embed(x3, table, pe):
    mesh = plsc.VectorSubcoreMesh(core_axis_name="c", subcore_axis_name="s")
    fn = functools.partial(
        pl.kernel,
        mesh=mesh,
        out_type=jax.ShapeDtypeStruct((BATCH * SEQ, D), jnp.float32),
        scratch_types=[
            pltpu.VMEM((ROWS_PER_W, 2, HALF), jnp.int32),
            pltpu.VMEM((CH * SEQ, D), jnp.float32),
            pltpu.VMEM((CH * SEQ, D), jnp.float32),
            pltpu.VMEM((CH * SEQ, D), jnp.float32),
            pltpu.VMEM((SEQ, D), jnp.float32),
            pltpu.SemaphoreType.DMA,
            pltpu.SemaphoreType.DMA,
        ],
        compiler_params=pltpu.CompilerParams(use_tc_tiling_on_sc=False),
    )(_sc_body)
    return fn(x3, table, pe)


def kernel(x, table):
    pe = _positional_encoding()
    x3 = x.reshape(BATCH, 2, HALF)
    return _embed(x3, table, pe).reshape(BATCH, SEQ, D)
